# baseline (speedup 1.0000x reference)
import jax
import jax.numpy as jnp
from jax.experimental import pallas as pl

N = 10000
HEADS = 8
HID = 32
NUM_CLASSES = 40


def _gat(x, edge_index, W, att_src, att_dst, bias, heads, out_ch, concat):
    n = x.shape[0]
    loop = jnp.arange(n, dtype=edge_index.dtype)
    src = jnp.concatenate([edge_index[0], loop])
    dst = jnp.concatenate([edge_index[1], loop])
    h = (x @ W).reshape(n, heads, out_ch)
    a_src = (h * att_src[None]).sum(-1)
    a_dst = (h * att_dst[None]).sum(-1)
    e = jax.nn.leaky_relu(a_src[src] + a_dst[dst], 0.2)
    m = jax.ops.segment_max(e, dst, num_segments=n)
    m = jnp.where(jnp.isfinite(m), m, 0.0)
    ex = jnp.exp(e - m[dst])
    denom = jax.ops.segment_sum(ex, dst, num_segments=n)
    alpha = ex / (denom[dst] + 1e-16)
    out = jax.ops.segment_sum(h[src] * alpha[..., None], dst, num_segments=n)
    if concat:
        out = out.reshape(n, heads * out_ch)
    else:
        out = out.mean(axis=1)
    return out + bias


def _bias_kernel(x_ref, b_ref, o_ref):
    o_ref[...] = x_ref[...] + b_ref[...]


def kernel(x, edge_index, batch, W1, att_src1, att_dst1, bias1, W2, att_src2, att_dst2, bias2):
    h = _gat(x, edge_index, W1, att_src1, att_dst1, jnp.zeros_like(bias1), HEADS, HID, True)
    h = h + bias1
    h = jax.nn.elu(h)
    out = _gat(h, edge_index, W2, att_src2, att_dst2, jnp.zeros_like(bias2), 1, NUM_CLASSES, False)
    out = pl.pallas_call(
        _bias_kernel,
        out_shape=jax.ShapeDtypeStruct(out.shape, out.dtype),
    )(out, jnp.broadcast_to(bias2[None, :], out.shape))
    return out


# speedup vs baseline: 16.8679x; 16.8679x over previous
import functools

import jax
import jax.numpy as jnp
from jax import lax
from jax.experimental import pallas as pl
from jax.experimental.pallas import tpu as pltpu
from jax.experimental.pallas import tpu_sc as plsc

N = 10000
NP = 10016
E0 = 320000
ET = E0 + N
EP = 331776
D_IN = 128
HEADS = 8
HID = 32
F1 = HEADS * HID
C2 = 40
C2P = 64
CHUNK_A = 1024
CHUNK_B = 2048
NW = 32

_MESH = plsc.VectorSubcoreMesh(core_axis_name="c", subcore_axis_name="s")
_SC_PARAMS = pltpu.CompilerParams(needs_layout_passes=False)


def _wid():
    return lax.axis_index("s") * 2 + lax.axis_index("c")


def _tc1_body(xT_ref, w1T_ref, amat_ref, h1T_ref, asadT_ref):
    h = jnp.dot(w1T_ref[...], xT_ref[...], preferred_element_type=jnp.float32)
    h1T_ref[...] = h
    asadT_ref[...] = jnp.dot(amat_ref[...], h, preferred_element_type=jnp.float32)


def _tc2_body(o1T_ref, b1_ref, w2T_ref, a2_ref, h2T_ref, asad2_ref):
    g = o1T_ref[...] + b1_ref[...]
    g = jnp.where(g > 0.0, g, jnp.exp(g) - 1.0)
    h2 = jnp.dot(w2T_ref[...], g, preferred_element_type=jnp.float32)
    h2T_ref[...] = h2
    asad2_ref[...] = jnp.dot(a2_ref[...], h2, preferred_element_type=jnp.float32)


def _sc_a1(asadT, srcH, dstH, zrow, exhT, denp,
           as_row, ad_row, den_row, srcb, dstb, exb):
    wid = _wid()
    k = wid % HEADS
    q = wid // HEADS
    pltpu.sync_copy(asadT.at[k], as_row)
    pltpu.sync_copy(asadT.at[k + HEADS], ad_row)
    pltpu.sync_copy(zrow.at[0], den_row)
    quarter = EP // 4
    base = q * quarter

    def chunk(j, _):
        off = base + j * CHUNK_A
        pltpu.sync_copy(srcH.at[pl.ds(off, CHUNK_A)], srcb)
        pltpu.sync_copy(dstH.at[pl.ds(off, CHUNK_A)], dstb)

        def grp(g, carry):
            s16 = srcb[pl.ds(g * 16, 16)]
            d16 = dstb[pl.ds(g * 16, 16)]
            e = plsc.load_gather(as_row, [s16]) + plsc.load_gather(ad_row, [d16])
            e = jnp.maximum(e, e * 0.2)
            ex = jnp.exp(e)
            exb[pl.ds(g * 16, 16)] = ex
            plsc.addupdate_scatter(den_row, [d16], ex)
            return carry

        lax.fori_loop(0, CHUNK_A // 16, grp, 0)
        pltpu.sync_copy(exb, exhT.at[k, pl.ds(off, CHUNK_A)])
        return _

    lax.fori_loop(0, quarter // CHUNK_A, chunk, 0)
    pltpu.sync_copy(den_row, denp.at[wid])


def _sc_a2(asad2, srcH, dstH, zrow, exh2, denp2,
           as_row, ad_row, den_row, srcb, dstb, exb):
    wid = _wid()
    span = EP // NW
    base = wid * span
    pltpu.sync_copy(asad2.at[0], as_row)
    pltpu.sync_copy(asad2.at[1], ad_row)
    pltpu.sync_copy(zrow.at[0], den_row)
    pltpu.sync_copy(srcH.at[pl.ds(base, span)], srcb)
    pltpu.sync_copy(dstH.at[pl.ds(base, span)], dstb)

    def grp(g, carry):
        s16 = srcb[pl.ds(g * 16, 16)]
        d16 = dstb[pl.ds(g * 16, 16)]
        e = plsc.load_gather(as_row, [s16]) + plsc.load_gather(ad_row, [d16])
        e = jnp.maximum(e, e * 0.2)
        ex = jnp.exp(e)
        exb[pl.ds(g * 16, 16)] = ex
        plsc.addupdate_scatter(den_row, [d16], ex)
        return carry

    lax.fori_loop(0, span // 16, grp, 0)
    pltpu.sync_copy(exb, exh2.at[pl.ds(base, span)])
    pltpu.sync_copy(den_row, denp2.at[wid])


def _rd_zero(rd_row):
    def z(i, c):
        rd_row[pl.ds(i * 16, 16)] = jnp.zeros((16,), jnp.float32)
        return c
    lax.fori_loop(0, NP // 16, z, 0)


def _rd_accum(rd_row, tmp_row):
    def a(i, c):
        s = pl.ds(i * 16, 16)
        rd_row[s] = rd_row[s] + tmp_row[s]
        return c
    lax.fori_loop(0, NP // 16, a, 0)


def _rd_recip(rd_row):
    def r(i, c):
        s = pl.ds(i * 16, 16)
        rd_row[s] = 1.0 / (rd_row[s] + 1e-16)
        return c
    lax.fori_loop(0, NP // 16, r, 0)


def _sc_b1(h1T, exhT, denp, srcH, dstH, zrow, outT,
           hbuf, acc, rd_row, tmp_row, srcb, dstb, exb, *, half):
    wid = _wid()
    r0 = half * 128 + 4 * wid
    head = half * 4 + wid // 8
    pltpu.sync_copy(h1T.at[pl.ds(r0, 4)], hbuf)
    pltpu.sync_copy(zrow, acc)
    _rd_zero(rd_row)
    for p in range(4):
        pltpu.sync_copy(denp.at[p * HEADS + head], tmp_row)
        _rd_accum(rd_row, tmp_row)
    _rd_recip(rd_row)

    def chunk(j, _):
        off = j * CHUNK_B
        pltpu.sync_copy(srcH.at[pl.ds(off, CHUNK_B)], srcb)
        pltpu.sync_copy(dstH.at[pl.ds(off, CHUNK_B)], dstb)
        pltpu.sync_copy(exhT.at[head, pl.ds(off, CHUNK_B)], exb)

        def grp(g, carry):
            s16 = srcb[pl.ds(g * 16, 16)]
            d16 = dstb[pl.ds(g * 16, 16)]
            al = exb[pl.ds(g * 16, 16)] * plsc.load_gather(rd_row, [d16])
            for c in range(4):
                cvec = jnp.full((16,), c, jnp.int32)
                v = plsc.load_gather(hbuf, [cvec, s16])
                plsc.addupdate_scatter(acc, [cvec, d16], v * al)
            return carry

        lax.fori_loop(0, CHUNK_B // 16, grp, 0)
        return _

    lax.fori_loop(0, EP // CHUNK_B, chunk, 0)
    pltpu.sync_copy(acc, outT.at[pl.ds(4 * wid, 4)])


def _sc_b2(h2T, exh2, denp2, srcH, dstH, zrow, outT,
           hbuf, acc, rd_row, tmp_row, srcb, dstb, exb):
    wid = _wid()
    r0 = 2 * wid
    pltpu.sync_copy(h2T.at[pl.ds(r0, 2)], hbuf)
    pltpu.sync_copy(zrow.at[pl.ds(0, 2)], acc)
    _rd_zero(rd_row)
    for p in range(NW):
        pltpu.sync_copy(denp2.at[p], tmp_row)
        _rd_accum(rd_row, tmp_row)
    _rd_recip(rd_row)

    def chunk(j, _):
        off = j * CHUNK_B
        pltpu.sync_copy(srcH.at[pl.ds(off, CHUNK_B)], srcb)
        pltpu.sync_copy(dstH.at[pl.ds(off, CHUNK_B)], dstb)
        pltpu.sync_copy(exh2.at[pl.ds(off, CHUNK_B)], exb)

        def grp(g, carry):
            s16 = srcb[pl.ds(g * 16, 16)]
            d16 = dstb[pl.ds(g * 16, 16)]
            al = exb[pl.ds(g * 16, 16)] * plsc.load_gather(rd_row, [d16])
            for c in range(2):
                cvec = jnp.full((16,), c, jnp.int32)
                v = plsc.load_gather(hbuf, [cvec, s16])
                plsc.addupdate_scatter(acc, [cvec, d16], v * al)
            return carry

        lax.fori_loop(0, CHUNK_B // 16, grp, 0)
        return _

    lax.fori_loop(0, EP // CHUNK_B, chunk, 0)
    pltpu.sync_copy(acc, outT.at[pl.ds(r0, 2)])


_f32 = jnp.float32


def _sds(shape):
    return jax.ShapeDtypeStruct(shape, _f32)


def kernel(x, edge_index, batch, W1, att_src1, att_dst1, bias1,
           W2, att_src2, att_dst2, bias2):
    loop = jnp.arange(N, dtype=jnp.int32)
    pad = jnp.full((EP - ET,), NP - 1, jnp.int32)
    src = jnp.concatenate([edge_index[0].astype(jnp.int32), loop, pad])
    dst = jnp.concatenate([edge_index[1].astype(jnp.int32), loop, pad])

    xT = jnp.zeros((D_IN, NP), _f32).at[:, :N].set(x.T)
    W1T = W1.T
    eye8 = jnp.eye(HEADS, dtype=_f32)
    amat = jnp.concatenate([
        (eye8[:, :, None] * att_src1[None, :, :]).reshape(HEADS, F1),
        (eye8[:, :, None] * att_dst1[None, :, :]).reshape(HEADS, F1),
    ], axis=0)
    zrow = jnp.zeros((4, NP), _f32)

    h1T, asadT = pl.pallas_call(
        _tc1_body,
        out_shape=(_sds((F1, NP)), _sds((2 * HEADS, NP))),
    )(xT, W1T, amat)

    a1 = pl.kernel(
        _sc_a1, mesh=_MESH, compiler_params=_SC_PARAMS,
        out_type=(_sds((HEADS, EP)), _sds((NW, NP))),
        scratch_types=[
            pltpu.VMEM((NP,), _f32), pltpu.VMEM((NP,), _f32),
            pltpu.VMEM((NP,), _f32),
            pltpu.VMEM((CHUNK_A,), jnp.int32), pltpu.VMEM((CHUNK_A,), jnp.int32),
            pltpu.VMEM((CHUNK_A,), _f32),
        ],
    )
    exhT, denp = a1(asadT, src, dst, zrow)

    def b1(half):
        return pl.kernel(
            functools.partial(_sc_b1, half=half), mesh=_MESH, compiler_params=_SC_PARAMS,
            out_type=_sds((128, NP)),
            scratch_types=[
                pltpu.VMEM((4, NP), _f32), pltpu.VMEM((4, NP), _f32),
                pltpu.VMEM((NP,), _f32), pltpu.VMEM((NP,), _f32),
                pltpu.VMEM((CHUNK_B,), jnp.int32),
                pltpu.VMEM((CHUNK_B,), jnp.int32),
                pltpu.VMEM((CHUNK_B,), _f32),
            ],
        )(h1T, exhT, denp, src, dst, zrow)

    o1T = jnp.concatenate([b1(0), b1(1)], axis=0)

    W2Tp = jnp.zeros((C2P, F1), _f32).at[:C2].set(W2.T)
    a2mat = jnp.zeros((2, C2P), _f32).at[0, :C2].set(att_src2[0]).at[1, :C2].set(att_dst2[0])
    h2T, asad2 = pl.pallas_call(
        _tc2_body,
        out_shape=(_sds((C2P, NP)), _sds((2, NP))),
    )(o1T, bias1.reshape(F1, 1), W2Tp, a2mat)

    a2 = pl.kernel(
        _sc_a2, mesh=_MESH, compiler_params=_SC_PARAMS,
        out_type=(_sds((EP,)), _sds((NW, NP))),
        scratch_types=[
            pltpu.VMEM((NP,), _f32), pltpu.VMEM((NP,), _f32),
            pltpu.VMEM((NP,), _f32),
            pltpu.VMEM((EP // NW,), jnp.int32), pltpu.VMEM((EP // NW,), jnp.int32),
            pltpu.VMEM((EP // NW,), _f32),
        ],
    )
    exh2, denp2 = a2(asad2, src, dst, zrow)

    o2T = pl.kernel(
        _sc_b2, mesh=_MESH, compiler_params=_SC_PARAMS,
        out_type=_sds((C2P, NP)),
        scratch_types=[
            pltpu.VMEM((2, NP), _f32), pltpu.VMEM((2, NP), _f32),
            pltpu.VMEM((NP,), _f32), pltpu.VMEM((NP,), _f32),
            pltpu.VMEM((CHUNK_B,), jnp.int32), pltpu.VMEM((CHUNK_B,), jnp.int32),
            pltpu.VMEM((CHUNK_B,), _f32),
        ],
    )(h2T, exh2, denp2, src, dst, zrow)

    return o2T[:C2, :N].T + bias2[None, :]


# speedup vs baseline: 24.8284x; 1.4719x over previous
import functools

import jax
import jax.numpy as jnp
from jax import lax
from jax.experimental import pallas as pl
from jax.experimental.pallas import tpu as pltpu
from jax.experimental.pallas import tpu_sc as plsc

N = 10000
NP = 10016
E0 = 320000
ET = E0 + N
EP = 331776
D_IN = 128
HEADS = 8
HID = 32
F1 = HEADS * HID
C2 = 40
C2P = 64
CHUNK_A = 1152
CHUNK_B = 2048
NW = 32

_MESH = plsc.VectorSubcoreMesh(core_axis_name="c", subcore_axis_name="s")
_SC_PARAMS = pltpu.CompilerParams(needs_layout_passes=False)


def _wid():
    return lax.axis_index("s") * 2 + lax.axis_index("c")


def _tc1_body(xT_ref, w1T_ref, amat_ref, h1T_ref, asadT_ref):
    h = jnp.dot(w1T_ref[...], xT_ref[...], preferred_element_type=jnp.float32)
    h1T_ref[...] = h
    asadT_ref[...] = jnp.dot(amat_ref[...], h, preferred_element_type=jnp.float32)


def _tc2_body(o1T_ref, b1_ref, w2T_ref, a2_ref, h2T_ref, asad2_ref):
    g = o1T_ref[...] + b1_ref[...]
    g = jnp.where(g > 0.0, g, jnp.exp(g) - 1.0)
    h2 = jnp.dot(w2T_ref[...], g, preferred_element_type=jnp.float32)
    h2T_ref[...] = h2
    asad2_ref[...] = jnp.dot(a2_ref[...], h2, preferred_element_type=jnp.float32)


def _row_zero(row):
    def z(i, c):
        row[pl.ds(i * 16, 16)] = jnp.zeros((16,), jnp.float32)
        return c
    lax.fori_loop(0, NP // 16, z, 0)


def _row_accum(row, tmp_row):
    def a(i, c):
        s = pl.ds(i * 16, 16)
        row[s] = row[s] + tmp_row[s]
        return c
    lax.fori_loop(0, NP // 16, a, 0)


def _row_recip(row):
    def r(i, c):
        s = pl.ds(i * 16, 16)
        row[s] = 1.0 / (row[s] + 1e-16)
        return c
    lax.fori_loop(0, NP // 16, r, 0)


def _db_loop(nchunks, issue, drain, body):
    issue(0, 0)
    issue(1, 1)

    def pair(jj, _):
        c0 = 2 * jj
        drain(0)
        body(0, c0)

        @pl.when(c0 + 2 < nchunks)
        def _i0():
            issue(c0 + 2, 0)

        drain(1)
        body(1, c0 + 1)

        @pl.when(c0 + 3 < nchunks)
        def _i1():
            issue(c0 + 3, 1)

        return _

    lax.fori_loop(0, nchunks // 2, pair, 0)


def _sc_a1(asadT, srcH, dstH, zrow, exhT, denp,
           as_row, ad_row, den_row,
           srcb0, dstb0, srcb1, dstb1, exb, sem0, sem1):
    wid = _wid()
    k = wid % HEADS
    q = wid // HEADS
    pltpu.sync_copy(asadT.at[k], as_row)
    pltpu.sync_copy(asadT.at[k + HEADS], ad_row)
    pltpu.sync_copy(zrow.at[0], den_row)
    quarter = EP // 4
    base = q * quarter
    bufs = ((srcb0, dstb0, sem0), (srcb1, dstb1, sem1))

    def issue(ci, slot):
        sb, db, sem = bufs[slot]
        off = base + ci * CHUNK_A
        pltpu.async_copy(srcH.at[pl.ds(off, CHUNK_A)], sb, sem)
        pltpu.async_copy(dstH.at[pl.ds(off, CHUNK_A)], db, sem)

    def drain(slot):
        sb, db, sem = bufs[slot]
        pltpu.make_async_copy(srcH.at[pl.ds(0, CHUNK_A)], sb, sem).wait()
        pltpu.make_async_copy(dstH.at[pl.ds(0, CHUNK_A)], db, sem).wait()

    def body(slot, ci):
        sb, db, _ = bufs[slot]

        def grp(g, carry):
            s16 = sb[pl.ds(g * 16, 16)]
            d16 = db[pl.ds(g * 16, 16)]
            e = plsc.load_gather(as_row, [s16]) + plsc.load_gather(ad_row, [d16])
            e = jnp.maximum(e, e * 0.2)
            ex = jnp.exp(e)
            exb[pl.ds(g * 16, 16)] = ex
            plsc.addupdate_scatter(den_row, [d16], ex)
            return carry

        lax.fori_loop(0, CHUNK_A // 16, grp, 0)
        pltpu.sync_copy(exb, exhT.at[k, pl.ds(base + ci * CHUNK_A, CHUNK_A)])

    _db_loop(quarter // CHUNK_A, issue, drain, body)
    pltpu.sync_copy(den_row, denp.at[wid])


def _sc_a2(asad2, srcH, dstH, zrow, exh2, denp2,
           as_row, ad_row, den_row, srcb, dstb, exb):
    wid = _wid()
    span = EP // NW
    base = wid * span
    pltpu.sync_copy(asad2.at[0], as_row)
    pltpu.sync_copy(asad2.at[1], ad_row)
    pltpu.sync_copy(zrow.at[0], den_row)
    pltpu.sync_copy(srcH.at[pl.ds(base, span)], srcb)
    pltpu.sync_copy(dstH.at[pl.ds(base, span)], dstb)

    def grp(g, carry):
        s16 = srcb[pl.ds(g * 16, 16)]
        d16 = dstb[pl.ds(g * 16, 16)]
        e = plsc.load_gather(as_row, [s16]) + plsc.load_gather(ad_row, [d16])
        e = jnp.maximum(e, e * 0.2)
        ex = jnp.exp(e)
        exb[pl.ds(g * 16, 16)] = ex
        plsc.addupdate_scatter(den_row, [d16], ex)
        return carry

    lax.fori_loop(0, span // 16, grp, 0)
    pltpu.sync_copy(exb, exh2.at[pl.ds(base, span)])
    pltpu.sync_copy(den_row, denp2.at[wid])


def _sc_b1(h1T, exhT, denp, srcH, dstH, zrow, outT,
           hbuf, acc, rd_row, tmp_row,
           srcb0, dstb0, exb0, srcb1, dstb1, exb1, sem0, sem1, *, half):
    wid = _wid()
    r0 = half * 128 + 4 * wid
    head = half * 4 + wid // 8
    pltpu.sync_copy(h1T.at[pl.ds(r0, 4)], hbuf)
    pltpu.sync_copy(zrow, acc)
    _row_zero(rd_row)
    for p in range(4):
        pltpu.sync_copy(denp.at[p * HEADS + head], tmp_row)
        _row_accum(rd_row, tmp_row)
    _row_recip(rd_row)
    bufs = ((srcb0, dstb0, exb0, sem0), (srcb1, dstb1, exb1, sem1))

    def issue(ci, slot):
        sb, db, eb, sem = bufs[slot]
        off = ci * CHUNK_B
        pltpu.async_copy(srcH.at[pl.ds(off, CHUNK_B)], sb, sem)
        pltpu.async_copy(dstH.at[pl.ds(off, CHUNK_B)], db, sem)
        pltpu.async_copy(exhT.at[head, pl.ds(off, CHUNK_B)], eb, sem)

    def drain(slot):
        sb, db, eb, sem = bufs[slot]
        pltpu.make_async_copy(srcH.at[pl.ds(0, CHUNK_B)], sb, sem).wait()
        pltpu.make_async_copy(dstH.at[pl.ds(0, CHUNK_B)], db, sem).wait()
        pltpu.make_async_copy(exhT.at[head, pl.ds(0, CHUNK_B)], eb, sem).wait()

    def body(slot, ci):
        sb, db, eb, _ = bufs[slot]

        def grp(g, carry):
            s16 = sb[pl.ds(g * 16, 16)]
            d16 = db[pl.ds(g * 16, 16)]
            al = eb[pl.ds(g * 16, 16)] * plsc.load_gather(rd_row, [d16])
            for c in range(4):
                cvec = jnp.full((16,), c, jnp.int32)
                v = plsc.load_gather(hbuf, [cvec, s16])
                plsc.addupdate_scatter(acc, [cvec, d16], v * al)
            return carry

        lax.fori_loop(0, CHUNK_B // 16, grp, 0)

    _db_loop(EP // CHUNK_B, issue, drain, body)
    pltpu.sync_copy(acc, outT.at[pl.ds(4 * wid, 4)])


def _sc_b2(h2T, exh2, denp2, srcH, dstH, zrow, outT,
           hbuf, acc, rd_row, tmp_row,
           srcb0, dstb0, exb0, srcb1, dstb1, exb1, sem0, sem1):
    wid = _wid()
    r0 = 2 * wid
    pltpu.sync_copy(h2T.at[pl.ds(r0, 2)], hbuf)
    pltpu.sync_copy(zrow.at[pl.ds(0, 2)], acc)
    _row_zero(rd_row)
    for p in range(NW):
        pltpu.sync_copy(denp2.at[p], tmp_row)
        _row_accum(rd_row, tmp_row)
    _row_recip(rd_row)
    bufs = ((srcb0, dstb0, exb0, sem0), (srcb1, dstb1, exb1, sem1))

    def issue(ci, slot):
        sb, db, eb, sem = bufs[slot]
        off = ci * CHUNK_B
        pltpu.async_copy(srcH.at[pl.ds(off, CHUNK_B)], sb, sem)
        pltpu.async_copy(dstH.at[pl.ds(off, CHUNK_B)], db, sem)
        pltpu.async_copy(exh2.at[pl.ds(off, CHUNK_B)], eb, sem)

    def drain(slot):
        sb, db, eb, sem = bufs[slot]
        pltpu.make_async_copy(srcH.at[pl.ds(0, CHUNK_B)], sb, sem).wait()
        pltpu.make_async_copy(dstH.at[pl.ds(0, CHUNK_B)], db, sem).wait()
        pltpu.make_async_copy(exh2.at[pl.ds(0, CHUNK_B)], eb, sem).wait()

    def body(slot, ci):
        sb, db, eb, _ = bufs[slot]

        def grp(g, carry):
            s16 = sb[pl.ds(g * 16, 16)]
            d16 = db[pl.ds(g * 16, 16)]
            al = eb[pl.ds(g * 16, 16)] * plsc.load_gather(rd_row, [d16])
            for c in range(2):
                cvec = jnp.full((16,), c, jnp.int32)
                v = plsc.load_gather(hbuf, [cvec, s16])
                plsc.addupdate_scatter(acc, [cvec, d16], v * al)
            return carry

        lax.fori_loop(0, CHUNK_B // 16, grp, 0)

    _db_loop(EP // CHUNK_B, issue, drain, body)
    pltpu.sync_copy(acc, outT.at[pl.ds(r0, 2)])


_f32 = jnp.float32


def _sds(shape):
    return jax.ShapeDtypeStruct(shape, _f32)


def kernel(x, edge_index, batch, W1, att_src1, att_dst1, bias1,
           W2, att_src2, att_dst2, bias2):
    loop = jnp.arange(N, dtype=jnp.int32)
    pad = jnp.full((EP - ET,), NP - 1, jnp.int32)
    src = jnp.concatenate([edge_index[0].astype(jnp.int32), loop, pad])
    dst = jnp.concatenate([edge_index[1].astype(jnp.int32), loop, pad])

    xT = jnp.zeros((D_IN, NP), _f32).at[:, :N].set(x.T)
    W1T = W1.T
    eye8 = jnp.eye(HEADS, dtype=_f32)
    amat = jnp.concatenate([
        (eye8[:, :, None] * att_src1[None, :, :]).reshape(HEADS, F1),
        (eye8[:, :, None] * att_dst1[None, :, :]).reshape(HEADS, F1),
    ], axis=0)
    zrow = jnp.zeros((4, NP), _f32)

    h1T, asadT = pl.pallas_call(
        _tc1_body,
        out_shape=(_sds((F1, NP)), _sds((2 * HEADS, NP))),
    )(xT, W1T, amat)

    a1 = pl.kernel(
        _sc_a1, mesh=_MESH, compiler_params=_SC_PARAMS,
        out_type=(_sds((HEADS, EP)), _sds((NW, NP))),
        scratch_types=[
            pltpu.VMEM((NP,), _f32), pltpu.VMEM((NP,), _f32),
            pltpu.VMEM((NP,), _f32),
            pltpu.VMEM((CHUNK_A,), jnp.int32), pltpu.VMEM((CHUNK_A,), jnp.int32),
            pltpu.VMEM((CHUNK_A,), jnp.int32), pltpu.VMEM((CHUNK_A,), jnp.int32),
            pltpu.VMEM((CHUNK_A,), _f32),
            pltpu.SemaphoreType.DMA, pltpu.SemaphoreType.DMA,
        ],
    )
    exhT, denp = a1(asadT, src, dst, zrow)

    def b1(half):
        return pl.kernel(
            functools.partial(_sc_b1, half=half), mesh=_MESH,
            compiler_params=_SC_PARAMS,
            out_type=_sds((128, NP)),
            scratch_types=[
                pltpu.VMEM((4, NP), _f32), pltpu.VMEM((4, NP), _f32),
                pltpu.VMEM((NP,), _f32), pltpu.VMEM((NP,), _f32),
                pltpu.VMEM((CHUNK_B,), jnp.int32),
                pltpu.VMEM((CHUNK_B,), jnp.int32),
                pltpu.VMEM((CHUNK_B,), _f32),
                pltpu.VMEM((CHUNK_B,), jnp.int32),
                pltpu.VMEM((CHUNK_B,), jnp.int32),
                pltpu.VMEM((CHUNK_B,), _f32),
                pltpu.SemaphoreType.DMA, pltpu.SemaphoreType.DMA,
            ],
        )(h1T, exhT, denp, src, dst, zrow)

    o1T = jnp.concatenate([b1(0), b1(1)], axis=0)

    W2Tp = jnp.zeros((C2P, F1), _f32).at[:C2].set(W2.T)
    a2mat = jnp.zeros((2, C2P), _f32).at[0, :C2].set(att_src2[0]).at[1, :C2].set(att_dst2[0])
    h2T, asad2 = pl.pallas_call(
        _tc2_body,
        out_shape=(_sds((C2P, NP)), _sds((2, NP))),
    )(o1T, bias1.reshape(F1, 1), W2Tp, a2mat)

    a2 = pl.kernel(
        _sc_a2, mesh=_MESH, compiler_params=_SC_PARAMS,
        out_type=(_sds((EP,)), _sds((NW, NP))),
        scratch_types=[
            pltpu.VMEM((NP,), _f32), pltpu.VMEM((NP,), _f32),
            pltpu.VMEM((NP,), _f32),
            pltpu.VMEM((EP // NW,), jnp.int32), pltpu.VMEM((EP // NW,), jnp.int32),
            pltpu.VMEM((EP // NW,), _f32),
        ],
    )
    exh2, denp2 = a2(asad2, src, dst, zrow)

    o2T = pl.kernel(
        _sc_b2, mesh=_MESH, compiler_params=_SC_PARAMS,
        out_type=_sds((C2P, NP)),
        scratch_types=[
            pltpu.VMEM((2, NP), _f32), pltpu.VMEM((2, NP), _f32),
            pltpu.VMEM((NP,), _f32), pltpu.VMEM((NP,), _f32),
            pltpu.VMEM((CHUNK_B,), jnp.int32), pltpu.VMEM((CHUNK_B,), jnp.int32),
            pltpu.VMEM((CHUNK_B,), _f32),
            pltpu.VMEM((CHUNK_B,), jnp.int32), pltpu.VMEM((CHUNK_B,), jnp.int32),
            pltpu.VMEM((CHUNK_B,), _f32),
            pltpu.SemaphoreType.DMA, pltpu.SemaphoreType.DMA,
        ],
    )(h2T, exh2, denp2, src, dst, zrow)

    return o2T[:C2, :N].T + bias2[None, :]


# speedup vs baseline: 26.1686x; 1.0540x over previous
import functools

import jax
import jax.numpy as jnp
from jax import lax
from jax.experimental import pallas as pl
from jax.experimental.pallas import tpu as pltpu
from jax.experimental.pallas import tpu_sc as plsc

N = 10000
NP = 10016
E0 = 320000
ET = E0 + N
EP = 331776
D_IN = 128
HEADS = 8
HID = 32
F1 = HEADS * HID
C2 = 40
C2P = 64
CHUNK_A = 1152
CHUNK_B = 2048
NW = 32

_MESH = plsc.VectorSubcoreMesh(core_axis_name="c", subcore_axis_name="s")
_SC_PARAMS = pltpu.CompilerParams(needs_layout_passes=False)


def _wid():
    return lax.axis_index("s") * 2 + lax.axis_index("c")


def _tc1_body(xT_ref, w1T_ref, amat_ref, h1T_ref, asadT_ref):
    h = jnp.dot(w1T_ref[...], xT_ref[...], preferred_element_type=jnp.float32)
    h1T_ref[...] = h
    asadT_ref[...] = jnp.dot(amat_ref[...], h, preferred_element_type=jnp.float32)


def _tc2_body(o1T_ref, b1_ref, w2T_ref, a2_ref, h2T_ref, asad2_ref):
    g = o1T_ref[...] + b1_ref[...]
    g = jnp.where(g > 0.0, g, jnp.exp(g) - 1.0)
    h2 = jnp.dot(w2T_ref[...], g, preferred_element_type=jnp.float32)
    h2T_ref[...] = h2
    asad2_ref[...] = jnp.dot(a2_ref[...], h2, preferred_element_type=jnp.float32)


def _row_zero(row):
    def z(i, c):
        row[pl.ds(i * 16, 16)] = jnp.zeros((16,), jnp.float32)
        return c
    lax.fori_loop(0, NP // 16, z, 0)


def _row_accum(row, tmp_row):
    def a(i, c):
        s = pl.ds(i * 16, 16)
        row[s] = row[s] + tmp_row[s]
        return c
    lax.fori_loop(0, NP // 16, a, 0)


def _row_recip(row):
    def r(i, c):
        s = pl.ds(i * 16, 16)
        row[s] = 1.0 / (row[s] + 1e-16)
        return c
    lax.fori_loop(0, NP // 16, r, 0)


def _db_loop(nchunks, issue, drain, body):
    issue(0, 0)
    issue(1, 1)

    def pair(jj, _):
        c0 = 2 * jj
        drain(0)
        body(0, c0)

        @pl.when(c0 + 2 < nchunks)
        def _i0():
            issue(c0 + 2, 0)

        drain(1)
        body(1, c0 + 1)

        @pl.when(c0 + 3 < nchunks)
        def _i1():
            issue(c0 + 3, 1)

        return _

    lax.fori_loop(0, nchunks // 2, pair, 0)


def _sc_a1(asadT, srcH, dstH, zrow, exhT, denp,
           as_row, ad_row, den_row,
           srcb0, dstb0, srcb1, dstb1, exb, sem0, sem1):
    wid = _wid()
    k = wid % HEADS
    q = wid // HEADS
    pltpu.sync_copy(asadT.at[k], as_row)
    pltpu.sync_copy(asadT.at[k + HEADS], ad_row)
    pltpu.sync_copy(zrow.at[0], den_row)
    quarter = EP // 4
    base = q * quarter
    bufs = ((srcb0, dstb0, sem0), (srcb1, dstb1, sem1))

    def issue(ci, slot):
        sb, db, sem = bufs[slot]
        off = base + ci * CHUNK_A
        pltpu.async_copy(srcH.at[pl.ds(off, CHUNK_A)], sb, sem)
        pltpu.async_copy(dstH.at[pl.ds(off, CHUNK_A)], db, sem)

    def drain(slot):
        sb, db, sem = bufs[slot]
        pltpu.make_async_copy(srcH.at[pl.ds(0, CHUNK_A)], sb, sem).wait()
        pltpu.make_async_copy(dstH.at[pl.ds(0, CHUNK_A)], db, sem).wait()

    def body(slot, ci):
        sb, db, _ = bufs[slot]

        def grp(g, carry):
            s16 = sb[pl.ds(g * 16, 16)]
            d16 = db[pl.ds(g * 16, 16)]
            e = plsc.load_gather(as_row, [s16]) + plsc.load_gather(ad_row, [d16])
            e = jnp.maximum(e, e * 0.2)
            ex = jnp.exp(e)
            exb[pl.ds(g * 16, 16)] = ex
            plsc.addupdate_scatter(den_row, [d16], ex)
            return carry

        lax.fori_loop(0, CHUNK_A // 16, grp, 0)
        pltpu.sync_copy(exb, exhT.at[k, pl.ds(base + ci * CHUNK_A, CHUNK_A)])

    _db_loop(quarter // CHUNK_A, issue, drain, body)
    pltpu.sync_copy(den_row, denp.at[wid])


def _sc_a2(asad2, srcH, dstH, zrow, exh2, denp2,
           as_row, ad_row, den_row, srcb, dstb, exb):
    wid = _wid()
    span = EP // NW
    base = wid * span
    pltpu.sync_copy(asad2.at[0], as_row)
    pltpu.sync_copy(asad2.at[1], ad_row)
    pltpu.sync_copy(zrow.at[0], den_row)
    pltpu.sync_copy(srcH.at[pl.ds(base, span)], srcb)
    pltpu.sync_copy(dstH.at[pl.ds(base, span)], dstb)

    def grp(g, carry):
        s16 = srcb[pl.ds(g * 16, 16)]
        d16 = dstb[pl.ds(g * 16, 16)]
        e = plsc.load_gather(as_row, [s16]) + plsc.load_gather(ad_row, [d16])
        e = jnp.maximum(e, e * 0.2)
        ex = jnp.exp(e)
        exb[pl.ds(g * 16, 16)] = ex
        plsc.addupdate_scatter(den_row, [d16], ex)
        return carry

    lax.fori_loop(0, span // 16, grp, 0)
    pltpu.sync_copy(exb, exh2.at[pl.ds(base, span)])
    pltpu.sync_copy(den_row, denp2.at[wid])


_UNROLL = 4


def _row_scale(row, rd_row):
    def m(i, c):
        s = pl.ds(i * 16, 16)
        row[s] = row[s] * rd_row[s]
        return c
    lax.fori_loop(0, NP // 16, m, 0)


def _b_scan(nrows, hrows, accs, bufs):
    def issue_drain_body(srcH, dstH, ex_at):
        def issue(ci, slot):
            sb, db, eb, sem = bufs[slot]
            off = ci * CHUNK_B
            pltpu.async_copy(srcH.at[pl.ds(off, CHUNK_B)], sb, sem)
            pltpu.async_copy(dstH.at[pl.ds(off, CHUNK_B)], db, sem)
            pltpu.async_copy(ex_at(off), eb, sem)

        def drain(slot):
            sb, db, eb, sem = bufs[slot]
            pltpu.make_async_copy(srcH.at[pl.ds(0, CHUNK_B)], sb, sem).wait()
            pltpu.make_async_copy(dstH.at[pl.ds(0, CHUNK_B)], db, sem).wait()
            pltpu.make_async_copy(ex_at(0), eb, sem).wait()

        def body(slot, ci):
            sb, db, eb, _ = bufs[slot]

            def grp(g, carry):
                for u in range(_UNROLL):
                    o = pl.ds(g * 16 * _UNROLL + u * 16, 16)
                    s16 = sb[o]
                    d16 = db[o]
                    al = eb[o]
                    for c in range(nrows):
                        v = plsc.load_gather(hrows[c], [s16])
                        plsc.addupdate_scatter(accs[c], [d16], v * al)
                return carry

            lax.fori_loop(0, CHUNK_B // (16 * _UNROLL), grp, 0)

        return issue, drain, body

    return issue_drain_body


def _sc_b1(h1T, exhT, denp, srcH, dstH, zrow, outT,
           hr0, hr1, hr2, hr3, ac0, ac1, ac2, ac3, rd_row, tmp_row,
           srcb0, dstb0, exb0, srcb1, dstb1, exb1, sem0, sem1, *, half):
    wid = _wid()
    r0 = half * 128 + 4 * wid
    head = half * 4 + wid // 8
    hrows = (hr0, hr1, hr2, hr3)
    accs = (ac0, ac1, ac2, ac3)
    for c in range(4):
        pltpu.sync_copy(h1T.at[r0 + c], hrows[c])
        pltpu.sync_copy(zrow.at[c], accs[c])
    bufs = ((srcb0, dstb0, exb0, sem0), (srcb1, dstb1, exb1, sem1))

    issue, drain, body = _b_scan(4, hrows, accs, bufs)(
        srcH, dstH, lambda off: exhT.at[head, pl.ds(off, CHUNK_B)])
    _db_loop(EP // CHUNK_B, issue, drain, body)

    _row_zero(rd_row)
    for p in range(4):
        pltpu.sync_copy(denp.at[p * HEADS + head], tmp_row)
        _row_accum(rd_row, tmp_row)
    _row_recip(rd_row)
    for c in range(4):
        _row_scale(accs[c], rd_row)
        pltpu.sync_copy(accs[c], outT.at[4 * wid + c])


def _sc_b2(h2T, exh2, denp2, srcH, dstH, zrow, outT,
           hr0, hr1, ac0, ac1, rd_row, tmp_row,
           srcb0, dstb0, exb0, srcb1, dstb1, exb1, sem0, sem1):
    wid = _wid()
    r0 = 2 * wid
    hrows = (hr0, hr1)
    accs = (ac0, ac1)
    for c in range(2):
        pltpu.sync_copy(h2T.at[r0 + c], hrows[c])
        pltpu.sync_copy(zrow.at[c], accs[c])
    bufs = ((srcb0, dstb0, exb0, sem0), (srcb1, dstb1, exb1, sem1))

    issue, drain, body = _b_scan(2, hrows, accs, bufs)(
        srcH, dstH, lambda off: exh2.at[pl.ds(off, CHUNK_B)])
    _db_loop(EP // CHUNK_B, issue, drain, body)

    _row_zero(rd_row)
    for p in range(NW):
        pltpu.sync_copy(denp2.at[p], tmp_row)
        _row_accum(rd_row, tmp_row)
    _row_recip(rd_row)
    for c in range(2):
        _row_scale(accs[c], rd_row)
        pltpu.sync_copy(accs[c], outT.at[r0 + c])


_f32 = jnp.float32


def _sds(shape):
    return jax.ShapeDtypeStruct(shape, _f32)


def kernel(x, edge_index, batch, W1, att_src1, att_dst1, bias1,
           W2, att_src2, att_dst2, bias2):
    loop = jnp.arange(N, dtype=jnp.int32)
    pad = jnp.full((EP - ET,), NP - 1, jnp.int32)
    src = jnp.concatenate([edge_index[0].astype(jnp.int32), loop, pad])
    dst = jnp.concatenate([edge_index[1].astype(jnp.int32), loop, pad])

    xT = jnp.zeros((D_IN, NP), _f32).at[:, :N].set(x.T)
    W1T = W1.T
    eye8 = jnp.eye(HEADS, dtype=_f32)
    amat = jnp.concatenate([
        (eye8[:, :, None] * att_src1[None, :, :]).reshape(HEADS, F1),
        (eye8[:, :, None] * att_dst1[None, :, :]).reshape(HEADS, F1),
    ], axis=0)
    zrow = jnp.zeros((4, NP), _f32)

    h1T, asadT = pl.pallas_call(
        _tc1_body,
        out_shape=(_sds((F1, NP)), _sds((2 * HEADS, NP))),
    )(xT, W1T, amat)

    a1 = pl.kernel(
        _sc_a1, mesh=_MESH, compiler_params=_SC_PARAMS,
        out_type=(_sds((HEADS, EP)), _sds((NW, NP))),
        scratch_types=[
            pltpu.VMEM((NP,), _f32), pltpu.VMEM((NP,), _f32),
            pltpu.VMEM((NP,), _f32),
            pltpu.VMEM((CHUNK_A,), jnp.int32), pltpu.VMEM((CHUNK_A,), jnp.int32),
            pltpu.VMEM((CHUNK_A,), jnp.int32), pltpu.VMEM((CHUNK_A,), jnp.int32),
            pltpu.VMEM((CHUNK_A,), _f32),
            pltpu.SemaphoreType.DMA, pltpu.SemaphoreType.DMA,
        ],
    )
    exhT, denp = a1(asadT, src, dst, zrow)

    def b1(half):
        return pl.kernel(
            functools.partial(_sc_b1, half=half), mesh=_MESH,
            compiler_params=_SC_PARAMS,
            out_type=_sds((128, NP)),
            scratch_types=[
                pltpu.VMEM((NP,), _f32), pltpu.VMEM((NP,), _f32),
                pltpu.VMEM((NP,), _f32), pltpu.VMEM((NP,), _f32),
                pltpu.VMEM((NP,), _f32), pltpu.VMEM((NP,), _f32),
                pltpu.VMEM((NP,), _f32), pltpu.VMEM((NP,), _f32),
                pltpu.VMEM((NP,), _f32), pltpu.VMEM((NP,), _f32),
                pltpu.VMEM((CHUNK_B,), jnp.int32),
                pltpu.VMEM((CHUNK_B,), jnp.int32),
                pltpu.VMEM((CHUNK_B,), _f32),
                pltpu.VMEM((CHUNK_B,), jnp.int32),
                pltpu.VMEM((CHUNK_B,), jnp.int32),
                pltpu.VMEM((CHUNK_B,), _f32),
                pltpu.SemaphoreType.DMA, pltpu.SemaphoreType.DMA,
            ],
        )(h1T, exhT, denp, src, dst, zrow)

    o1T = jnp.concatenate([b1(0), b1(1)], axis=0)

    W2Tp = jnp.zeros((C2P, F1), _f32).at[:C2].set(W2.T)
    a2mat = jnp.zeros((2, C2P), _f32).at[0, :C2].set(att_src2[0]).at[1, :C2].set(att_dst2[0])
    h2T, asad2 = pl.pallas_call(
        _tc2_body,
        out_shape=(_sds((C2P, NP)), _sds((2, NP))),
    )(o1T, bias1.reshape(F1, 1), W2Tp, a2mat)

    a2 = pl.kernel(
        _sc_a2, mesh=_MESH, compiler_params=_SC_PARAMS,
        out_type=(_sds((EP,)), _sds((NW, NP))),
        scratch_types=[
            pltpu.VMEM((NP,), _f32), pltpu.VMEM((NP,), _f32),
            pltpu.VMEM((NP,), _f32),
            pltpu.VMEM((EP // NW,), jnp.int32), pltpu.VMEM((EP // NW,), jnp.int32),
            pltpu.VMEM((EP // NW,), _f32),
        ],
    )
    exh2, denp2 = a2(asad2, src, dst, zrow)

    o2T = pl.kernel(
        _sc_b2, mesh=_MESH, compiler_params=_SC_PARAMS,
        out_type=_sds((C2P, NP)),
        scratch_types=[
            pltpu.VMEM((NP,), _f32), pltpu.VMEM((NP,), _f32),
            pltpu.VMEM((NP,), _f32), pltpu.VMEM((NP,), _f32),
            pltpu.VMEM((NP,), _f32), pltpu.VMEM((NP,), _f32),
            pltpu.VMEM((CHUNK_B,), jnp.int32), pltpu.VMEM((CHUNK_B,), jnp.int32),
            pltpu.VMEM((CHUNK_B,), _f32),
            pltpu.VMEM((CHUNK_B,), jnp.int32), pltpu.VMEM((CHUNK_B,), jnp.int32),
            pltpu.VMEM((CHUNK_B,), _f32),
            pltpu.SemaphoreType.DMA, pltpu.SemaphoreType.DMA,
        ],
    )(h2T, exh2, denp2, src, dst, zrow)

    return o2T[:C2, :N].T + bias2[None, :]


# speedup vs baseline: 31.2799x; 1.1953x over previous
import functools

import jax
import jax.numpy as jnp
from jax import lax
from jax.experimental import pallas as pl
from jax.experimental.pallas import tpu as pltpu
from jax.experimental.pallas import tpu_sc as plsc

N = 10000
NP = 10016
E0 = 320000
ET = E0 + N
EP = 331776
D_IN = 128
HEADS = 8
HID = 32
F1 = HEADS * HID
C2 = 40
C2P = 64
CHUNK_A = 1152
CHUNK_B = 2048
NW = 32
_UNROLL = 4

_MESH = plsc.VectorSubcoreMesh(core_axis_name="c", subcore_axis_name="s")
_SC_PARAMS = pltpu.CompilerParams(needs_layout_passes=False)
_f32 = jnp.float32
_bf16 = jnp.bfloat16


def _wid():
    return lax.axis_index("s") * 2 + lax.axis_index("c")


def _unpack_sd(sdv):
    s16 = jnp.bitwise_and(sdv, jnp.int32(0xFFFF))
    d16 = lax.shift_right_logical(sdv, 16)
    return s16, d16


def _unpack_pair(pw):
    return plsc.unpack(plsc.bitcast(pw, _bf16),
                       format=plsc.PackFormat.INTERLEAVED,
                       preferred_element_type=_f32)


def _tc1_body(xT_ref, w1T_ref, amat_ref, h1T_ref, asadT_ref):
    h = jnp.dot(w1T_ref[...], xT_ref[...], preferred_element_type=jnp.float32)
    h1T_ref[...] = h
    asadT_ref[...] = jnp.dot(amat_ref[...], h, preferred_element_type=jnp.float32)


def _tc2_body(o1T_ref, b1_ref, w2T_ref, a2_ref, h2T_ref, asad2_ref):
    g = o1T_ref[...] + b1_ref[...]
    g = jnp.where(g > 0.0, g, jnp.exp(g) - 1.0)
    h2 = jnp.dot(w2T_ref[...], g, preferred_element_type=jnp.float32)
    h2T_ref[...] = h2
    asad2_ref[...] = jnp.dot(a2_ref[...], h2, preferred_element_type=jnp.float32)


def _row_zero(row):
    def z(i, c):
        row[pl.ds(i * 16, 16)] = jnp.zeros((16,), jnp.float32)
        return c
    lax.fori_loop(0, NP // 16, z, 0)


def _row_accum(row, tmp_row):
    def a(i, c):
        s = pl.ds(i * 16, 16)
        row[s] = row[s] + tmp_row[s]
        return c
    lax.fori_loop(0, NP // 16, a, 0)


def _row_recip(row):
    def r(i, c):
        s = pl.ds(i * 16, 16)
        row[s] = 1.0 / (row[s] + 1e-16)
        return c
    lax.fori_loop(0, NP // 16, r, 0)


def _row_scale(row, rd_row):
    def m(i, c):
        s = pl.ds(i * 16, 16)
        row[s] = row[s] * rd_row[s]
        return c
    lax.fori_loop(0, NP // 16, m, 0)


def _db_loop(nchunks, issue, drain, body):
    issue(0, 0)
    issue(1, 1)

    def pair(jj, _):
        c0 = 2 * jj
        drain(0)
        body(0, c0)

        @pl.when(c0 + 2 < nchunks)
        def _i0():
            issue(c0 + 2, 0)

        drain(1)
        body(1, c0 + 1)

        @pl.when(c0 + 3 < nchunks)
        def _i1():
            issue(c0 + 3, 1)

        return _

    lax.fori_loop(0, nchunks // 2, pair, 0)


def _sc_a1(asadT, sdH, zrow, exhT, denp,
           as_row, ad_row, den_row,
           sdb0, sdb1, exb, sem0, sem1):
    wid = _wid()
    k = wid % HEADS
    q = wid // HEADS
    pltpu.sync_copy(asadT.at[k], as_row)
    pltpu.sync_copy(asadT.at[k + HEADS], ad_row)
    pltpu.sync_copy(zrow.at[0], den_row)
    quarter = EP // 4
    base = q * quarter
    bufs = ((sdb0, sem0), (sdb1, sem1))

    def issue(ci, slot):
        sb, sem = bufs[slot]
        off = base + ci * CHUNK_A
        pltpu.async_copy(sdH.at[pl.ds(off, CHUNK_A)], sb, sem)

    def drain(slot):
        sb, sem = bufs[slot]
        pltpu.make_async_copy(sdH.at[pl.ds(0, CHUNK_A)], sb, sem).wait()

    def body(slot, ci):
        sb, _ = bufs[slot]

        def grp(g, carry):
            s16, d16 = _unpack_sd(sb[pl.ds(g * 16, 16)])
            e = plsc.load_gather(as_row, [s16]) + plsc.load_gather(ad_row, [d16])
            e = jnp.maximum(e, e * 0.2)
            ex = jnp.exp(e)
            exb[pl.ds(g * 16, 16)] = ex
            plsc.addupdate_scatter(den_row, [d16], ex)
            return carry

        lax.fori_loop(0, CHUNK_A // 16, grp, 0)
        pltpu.sync_copy(exb, exhT.at[k, pl.ds(base + ci * CHUNK_A, CHUNK_A)])

    _db_loop(quarter // CHUNK_A, issue, drain, body)
    pltpu.sync_copy(den_row, denp.at[wid])


def _sc_a2(asad2, sdH, zrow, exh2, denp2,
           as_row, ad_row, den_row, sdb, exb):
    wid = _wid()
    span = EP // NW
    base = wid * span
    pltpu.sync_copy(asad2.at[0], as_row)
    pltpu.sync_copy(asad2.at[1], ad_row)
    pltpu.sync_copy(zrow.at[0], den_row)
    pltpu.sync_copy(sdH.at[pl.ds(base, span)], sdb)

    def grp(g, carry):
        s16, d16 = _unpack_sd(sdb[pl.ds(g * 16, 16)])
        e = plsc.load_gather(as_row, [s16]) + plsc.load_gather(ad_row, [d16])
        e = jnp.maximum(e, e * 0.2)
        ex = jnp.exp(e)
        exb[pl.ds(g * 16, 16)] = ex
        plsc.addupdate_scatter(den_row, [d16], ex)
        return carry

    lax.fori_loop(0, span // 16, grp, 0)
    pltpu.sync_copy(exb, exh2.at[pl.ds(base, span)])
    pltpu.sync_copy(den_row, denp2.at[wid])


def _b_scan(npk, hpk, accs, bufs, sdH, ex_at):
    def issue(ci, slot):
        sdb, eb, sem = bufs[slot]
        off = ci * CHUNK_B
        pltpu.async_copy(sdH.at[pl.ds(off, CHUNK_B)], sdb, sem)
        pltpu.async_copy(ex_at(off), eb, sem)

    def drain(slot):
        sdb, eb, sem = bufs[slot]
        pltpu.make_async_copy(sdH.at[pl.ds(0, CHUNK_B)], sdb, sem).wait()
        pltpu.make_async_copy(ex_at(0), eb, sem).wait()

    def body(slot, ci):
        sdb, eb, _ = bufs[slot]

        def grp(g, carry):
            for u in range(_UNROLL):
                o = pl.ds(g * 16 * _UNROLL + u * 16, 16)
                s16, d16 = _unpack_sd(sdb[o])
                al = eb[o]
                for p in range(npk):
                    pw = plsc.load_gather(hpk[p], [s16])
                    v0, v1 = _unpack_pair(pw)
                    plsc.addupdate_scatter(accs[2 * p], [d16], v0 * al)
                    plsc.addupdate_scatter(accs[2 * p + 1], [d16], v1 * al)
            return carry

        lax.fori_loop(0, CHUNK_B // (16 * _UNROLL), grp, 0)

    return issue, drain, body


def _sc_b1(h1p, exhT, denp, sdH, zrow, outT,
           hp0, hp1, ac0, ac1, ac2, ac3, rd_row, tmp_row,
           sdb0, exb0, sdb1, exb1, sem0, sem1, *, half):
    wid = _wid()
    head = half * 4 + wid // 8
    hpk = (hp0, hp1)
    accs = (ac0, ac1, ac2, ac3)
    for p in range(2):
        pltpu.sync_copy(h1p.at[half * 64 + 2 * wid + p], hpk[p])
    for c in range(4):
        pltpu.sync_copy(zrow.at[c], accs[c])
    bufs = ((sdb0, exb0, sem0), (sdb1, exb1, sem1))

    issue, drain, body = _b_scan(
        2, hpk, accs, bufs, sdH,
        lambda off: exhT.at[head, pl.ds(off, CHUNK_B)])
    _db_loop(EP // CHUNK_B, issue, drain, body)

    _row_zero(rd_row)
    for p in range(4):
        pltpu.sync_copy(denp.at[p * HEADS + head], tmp_row)
        _row_accum(rd_row, tmp_row)
    _row_recip(rd_row)
    for c in range(4):
        _row_scale(accs[c], rd_row)
        pltpu.sync_copy(accs[c], outT.at[4 * wid + c])


def _sc_b2(h2p, exh2, denp2, sdH, zrow, outT,
           hp0, ac0, ac1, rd_row, tmp_row,
           sdb0, exb0, sdb1, exb1, sem0, sem1):
    wid = _wid()
    accs = (ac0, ac1)
    pltpu.sync_copy(h2p.at[wid], hp0)
    for c in range(2):
        pltpu.sync_copy(zrow.at[c], accs[c])
    bufs = ((sdb0, exb0, sem0), (sdb1, exb1, sem1))

    issue, drain, body = _b_scan(
        1, (hp0,), accs, bufs, sdH,
        lambda off: exh2.at[pl.ds(off, CHUNK_B)])
    _db_loop(EP // CHUNK_B, issue, drain, body)

    _row_zero(rd_row)
    for p in range(NW):
        pltpu.sync_copy(denp2.at[p], tmp_row)
        _row_accum(rd_row, tmp_row)
    _row_recip(rd_row)
    for c in range(2):
        _row_scale(accs[c], rd_row)
        pltpu.sync_copy(accs[c], outT.at[2 * wid + c])


def _sds(shape, dtype=_f32):
    return jax.ShapeDtypeStruct(shape, dtype)


def _pack_rows(hT, nrow):
    b = hT.astype(_bf16).reshape(nrow, 2, NP).transpose(0, 2, 1)
    return lax.bitcast_convert_type(b, _f32)


def kernel(x, edge_index, batch, W1, att_src1, att_dst1, bias1,
           W2, att_src2, att_dst2, bias2):
    loop = jnp.arange(N, dtype=jnp.int32)
    pad = jnp.full((EP - ET,), NP - 1, jnp.int32)
    src = jnp.concatenate([edge_index[0].astype(jnp.int32), loop, pad])
    dst = jnp.concatenate([edge_index[1].astype(jnp.int32), loop, pad])
    sd = jnp.bitwise_or(src, lax.shift_left(dst, 16))

    xT = jnp.zeros((D_IN, NP), _f32).at[:, :N].set(x.T)
    W1T = W1.T
    eye8 = jnp.eye(HEADS, dtype=_f32)
    amat = jnp.concatenate([
        (eye8[:, :, None] * att_src1[None, :, :]).reshape(HEADS, F1),
        (eye8[:, :, None] * att_dst1[None, :, :]).reshape(HEADS, F1),
    ], axis=0)
    zrow = jnp.zeros((4, NP), _f32)

    h1T, asadT = pl.pallas_call(
        _tc1_body,
        out_shape=(_sds((F1, NP)), _sds((2 * HEADS, NP))),
    )(xT, W1T, amat)
    h1p = _pack_rows(h1T, F1 // 2)

    a1 = pl.kernel(
        _sc_a1, mesh=_MESH, compiler_params=_SC_PARAMS,
        out_type=(_sds((HEADS, EP)), _sds((NW, NP))),
        scratch_types=[
            pltpu.VMEM((NP,), _f32), pltpu.VMEM((NP,), _f32),
            pltpu.VMEM((NP,), _f32),
            pltpu.VMEM((CHUNK_A,), jnp.int32), pltpu.VMEM((CHUNK_A,), jnp.int32),
            pltpu.VMEM((CHUNK_A,), _f32),
            pltpu.SemaphoreType.DMA, pltpu.SemaphoreType.DMA,
        ],
    )
    exhT, denp = a1(asadT, sd, zrow)

    def b1(half):
        return pl.kernel(
            functools.partial(_sc_b1, half=half), mesh=_MESH,
            compiler_params=_SC_PARAMS,
            out_type=_sds((128, NP)),
            scratch_types=[
                pltpu.VMEM((NP,), _f32), pltpu.VMEM((NP,), _f32),
                pltpu.VMEM((NP,), _f32), pltpu.VMEM((NP,), _f32),
                pltpu.VMEM((NP,), _f32), pltpu.VMEM((NP,), _f32),
                pltpu.VMEM((NP,), _f32), pltpu.VMEM((NP,), _f32),
                pltpu.VMEM((CHUNK_B,), jnp.int32),
                pltpu.VMEM((CHUNK_B,), _f32),
                pltpu.VMEM((CHUNK_B,), jnp.int32),
                pltpu.VMEM((CHUNK_B,), _f32),
                pltpu.SemaphoreType.DMA, pltpu.SemaphoreType.DMA,
            ],
        )(h1p, exhT, denp, sd, zrow)

    o1T = jnp.concatenate([b1(0), b1(1)], axis=0)

    W2Tp = jnp.zeros((C2P, F1), _f32).at[:C2].set(W2.T)
    a2mat = jnp.zeros((2, C2P), _f32).at[0, :C2].set(att_src2[0]).at[1, :C2].set(att_dst2[0])
    h2T, asad2 = pl.pallas_call(
        _tc2_body,
        out_shape=(_sds((C2P, NP)), _sds((2, NP))),
    )(o1T, bias1.reshape(F1, 1), W2Tp, a2mat)
    h2p = _pack_rows(h2T, C2P // 2)

    a2 = pl.kernel(
        _sc_a2, mesh=_MESH, compiler_params=_SC_PARAMS,
        out_type=(_sds((EP,)), _sds((NW, NP))),
        scratch_types=[
            pltpu.VMEM((NP,), _f32), pltpu.VMEM((NP,), _f32),
            pltpu.VMEM((NP,), _f32),
            pltpu.VMEM((EP // NW,), jnp.int32),
            pltpu.VMEM((EP // NW,), _f32),
        ],
    )
    exh2, denp2 = a2(asad2, sd, zrow)

    o2T = pl.kernel(
        _sc_b2, mesh=_MESH, compiler_params=_SC_PARAMS,
        out_type=_sds((C2P, NP)),
        scratch_types=[
            pltpu.VMEM((NP,), _f32),
            pltpu.VMEM((NP,), _f32), pltpu.VMEM((NP,), _f32),
            pltpu.VMEM((NP,), _f32), pltpu.VMEM((NP,), _f32),
            pltpu.VMEM((CHUNK_B,), jnp.int32), pltpu.VMEM((CHUNK_B,), _f32),
            pltpu.VMEM((CHUNK_B,), jnp.int32), pltpu.VMEM((CHUNK_B,), _f32),
            pltpu.SemaphoreType.DMA, pltpu.SemaphoreType.DMA,
        ],
    )(h2p, exh2, denp2, sd, zrow)

    return o2T[:C2, :N].T + bias2[None, :]


# speedup vs baseline: 31.3147x; 1.0011x over previous
import functools

import jax
import jax.numpy as jnp
from jax import lax
from jax.experimental import pallas as pl
from jax.experimental.pallas import tpu as pltpu
from jax.experimental.pallas import tpu_sc as plsc

N = 10000
NP = 10016
E0 = 320000
ET = E0 + N
EP = 331776
D_IN = 128
HEADS = 8
HID = 32
F1 = HEADS * HID
C2 = 40
C2P = 64
CHUNK_A = 1152
CHUNK_B = 2048
NW = 32
_UNROLL = 8

_MESH = plsc.VectorSubcoreMesh(core_axis_name="c", subcore_axis_name="s")
_SC_PARAMS = pltpu.CompilerParams(needs_layout_passes=False)
_f32 = jnp.float32
_bf16 = jnp.bfloat16


def _wid():
    return lax.axis_index("s") * 2 + lax.axis_index("c")


def _unpack_sd(sdv):
    s16 = jnp.bitwise_and(sdv, jnp.int32(0xFFFF))
    d16 = lax.shift_right_logical(sdv, 16)
    return s16, d16


def _unpack_pair(pw):
    return plsc.unpack(plsc.bitcast(pw, _bf16),
                       format=plsc.PackFormat.INTERLEAVED,
                       preferred_element_type=_f32)


def _tc1_body(xT_ref, w1T_ref, amat_ref, h1T_ref, asadT_ref):
    h = jnp.dot(w1T_ref[...], xT_ref[...], preferred_element_type=jnp.float32)
    h1T_ref[...] = h
    asadT_ref[...] = jnp.dot(amat_ref[...], h, preferred_element_type=jnp.float32)


def _tc2_body(o1T_ref, b1_ref, w2T_ref, a2_ref, h2T_ref, asad2_ref):
    g = o1T_ref[...] + b1_ref[...]
    g = jnp.where(g > 0.0, g, jnp.exp(g) - 1.0)
    h2 = jnp.dot(w2T_ref[...], g, preferred_element_type=jnp.float32)
    h2T_ref[...] = h2
    asad2_ref[...] = jnp.dot(a2_ref[...], h2, preferred_element_type=jnp.float32)


def _row_zero(row):
    def z(i, c):
        row[pl.ds(i * 16, 16)] = jnp.zeros((16,), jnp.float32)
        return c
    lax.fori_loop(0, NP // 16, z, 0)


def _row_accum(row, tmp_row):
    def a(i, c):
        s = pl.ds(i * 16, 16)
        row[s] = row[s] + tmp_row[s]
        return c
    lax.fori_loop(0, NP // 16, a, 0)


def _row_recip(row):
    def r(i, c):
        s = pl.ds(i * 16, 16)
        row[s] = 1.0 / (row[s] + 1e-16)
        return c
    lax.fori_loop(0, NP // 16, r, 0)


def _row_scale(row, rd_row):
    def m(i, c):
        s = pl.ds(i * 16, 16)
        row[s] = row[s] * rd_row[s]
        return c
    lax.fori_loop(0, NP // 16, m, 0)


def _db_loop(nchunks, issue, drain, body):
    issue(0, 0)
    issue(1, 1)

    def pair(jj, _):
        c0 = 2 * jj
        drain(0)
        body(0, c0)

        @pl.when(c0 + 2 < nchunks)
        def _i0():
            issue(c0 + 2, 0)

        drain(1)
        body(1, c0 + 1)

        @pl.when(c0 + 3 < nchunks)
        def _i1():
            issue(c0 + 3, 1)

        return _

    lax.fori_loop(0, nchunks // 2, pair, 0)


def _sc_a1(asadT, sdH, zrow, exhT, denp,
           as_row, ad_row, den_row,
           sdb0, sdb1, exb, sem0, sem1):
    wid = _wid()
    k = wid % HEADS
    q = wid // HEADS
    pltpu.sync_copy(asadT.at[k], as_row)
    pltpu.sync_copy(asadT.at[k + HEADS], ad_row)
    pltpu.sync_copy(zrow.at[0], den_row)
    quarter = EP // 4
    base = q * quarter
    bufs = ((sdb0, sem0), (sdb1, sem1))

    def issue(ci, slot):
        sb, sem = bufs[slot]
        off = base + ci * CHUNK_A
        pltpu.async_copy(sdH.at[pl.ds(off, CHUNK_A)], sb, sem)

    def drain(slot):
        sb, sem = bufs[slot]
        pltpu.make_async_copy(sdH.at[pl.ds(0, CHUNK_A)], sb, sem).wait()

    def body(slot, ci):
        sb, _ = bufs[slot]

        def grp(g, carry):
            s16, d16 = _unpack_sd(sb[pl.ds(g * 16, 16)])
            e = plsc.load_gather(as_row, [s16]) + plsc.load_gather(ad_row, [d16])
            e = jnp.maximum(e, e * 0.2)
            ex = jnp.exp(e)
            exb[pl.ds(g * 16, 16)] = ex
            plsc.addupdate_scatter(den_row, [d16], ex)
            return carry

        lax.fori_loop(0, CHUNK_A // 16, grp, 0)
        pltpu.sync_copy(exb, exhT.at[k, pl.ds(base + ci * CHUNK_A, CHUNK_A)])

    _db_loop(quarter // CHUNK_A, issue, drain, body)
    pltpu.sync_copy(den_row, denp.at[wid])


def _sc_a2(asad2, sdH, zrow, exh2, denp2,
           as_row, ad_row, den_row, sdb, exb):
    wid = _wid()
    span = EP // NW
    base = wid * span
    pltpu.sync_copy(asad2.at[0], as_row)
    pltpu.sync_copy(asad2.at[1], ad_row)
    pltpu.sync_copy(zrow.at[0], den_row)
    pltpu.sync_copy(sdH.at[pl.ds(base, span)], sdb)

    def grp(g, carry):
        s16, d16 = _unpack_sd(sdb[pl.ds(g * 16, 16)])
        e = plsc.load_gather(as_row, [s16]) + plsc.load_gather(ad_row, [d16])
        e = jnp.maximum(e, e * 0.2)
        ex = jnp.exp(e)
        exb[pl.ds(g * 16, 16)] = ex
        plsc.addupdate_scatter(den_row, [d16], ex)
        return carry

    lax.fori_loop(0, span // 16, grp, 0)
    pltpu.sync_copy(exb, exh2.at[pl.ds(base, span)])
    pltpu.sync_copy(den_row, denp2.at[wid])


def _b_scan(npk, hpk, accs, bufs, sdH, ex_at):
    def issue(ci, slot):
        sdb, eb, sem = bufs[slot]
        off = ci * CHUNK_B
        pltpu.async_copy(sdH.at[pl.ds(off, CHUNK_B)], sdb, sem)
        pltpu.async_copy(ex_at(off), eb, sem)

    def drain(slot):
        sdb, eb, sem = bufs[slot]
        pltpu.make_async_copy(sdH.at[pl.ds(0, CHUNK_B)], sdb, sem).wait()
        pltpu.make_async_copy(ex_at(0), eb, sem).wait()

    def body(slot, ci):
        sdb, eb, _ = bufs[slot]

        def grp(g, carry):
            for u in range(_UNROLL):
                o = pl.ds(g * 16 * _UNROLL + u * 16, 16)
                s16, d16 = _unpack_sd(sdb[o])
                al = eb[o]
                for p in range(npk):
                    pw = plsc.load_gather(hpk[p], [s16])
                    v0, v1 = _unpack_pair(pw)
                    plsc.addupdate_scatter(accs[2 * p], [d16], v0 * al)
                    plsc.addupdate_scatter(accs[2 * p + 1], [d16], v1 * al)
            return carry

        lax.fori_loop(0, CHUNK_B // (16 * _UNROLL), grp, 0)

    return issue, drain, body


def _sc_b1(h1p, exhT, denp, sdH, zrow, outT,
           hp0, hp1, ac0, ac1, ac2, ac3, rd_row, tmp_row,
           sdb0, exb0, sdb1, exb1, sem0, sem1, *, half):
    wid = _wid()
    head = half * 4 + wid // 8
    hpk = (hp0, hp1)
    accs = (ac0, ac1, ac2, ac3)
    for p in range(2):
        pltpu.sync_copy(h1p.at[half * 64 + 2 * wid + p], hpk[p])
    for c in range(4):
        pltpu.sync_copy(zrow.at[c], accs[c])
    bufs = ((sdb0, exb0, sem0), (sdb1, exb1, sem1))

    issue, drain, body = _b_scan(
        2, hpk, accs, bufs, sdH,
        lambda off: exhT.at[head, pl.ds(off, CHUNK_B)])
    _db_loop(EP // CHUNK_B, issue, drain, body)

    _row_zero(rd_row)
    for p in range(4):
        pltpu.sync_copy(denp.at[p * HEADS + head], tmp_row)
        _row_accum(rd_row, tmp_row)
    _row_recip(rd_row)
    for c in range(4):
        _row_scale(accs[c], rd_row)
        pltpu.sync_copy(accs[c], outT.at[4 * wid + c])


def _sc_b2(h2p, exh2, denp2, sdH, zrow, outT,
           hp0, ac0, ac1, rd_row, tmp_row,
           sdb0, exb0, sdb1, exb1, sem0, sem1):
    wid = _wid()
    accs = (ac0, ac1)
    pltpu.sync_copy(h2p.at[wid], hp0)
    for c in range(2):
        pltpu.sync_copy(zrow.at[c], accs[c])
    bufs = ((sdb0, exb0, sem0), (sdb1, exb1, sem1))

    issue, drain, body = _b_scan(
        1, (hp0,), accs, bufs, sdH,
        lambda off: exh2.at[pl.ds(off, CHUNK_B)])
    _db_loop(EP // CHUNK_B, issue, drain, body)

    _row_zero(rd_row)
    for p in range(NW):
        pltpu.sync_copy(denp2.at[p], tmp_row)
        _row_accum(rd_row, tmp_row)
    _row_recip(rd_row)
    for c in range(2):
        _row_scale(accs[c], rd_row)
        pltpu.sync_copy(accs[c], outT.at[2 * wid + c])


def _sds(shape, dtype=_f32):
    return jax.ShapeDtypeStruct(shape, dtype)


def _pack_rows(hT, nrow):
    b = hT.astype(_bf16).reshape(nrow, 2, NP).transpose(0, 2, 1)
    return lax.bitcast_convert_type(b, _f32)


def kernel(x, edge_index, batch, W1, att_src1, att_dst1, bias1,
           W2, att_src2, att_dst2, bias2):
    loop = jnp.arange(N, dtype=jnp.int32)
    pad = jnp.full((EP - ET,), NP - 1, jnp.int32)
    src = jnp.concatenate([edge_index[0].astype(jnp.int32), loop, pad])
    dst = jnp.concatenate([edge_index[1].astype(jnp.int32), loop, pad])
    sd = jnp.bitwise_or(src, lax.shift_left(dst, 16))

    xT = jnp.zeros((D_IN, NP), _f32).at[:, :N].set(x.T)
    W1T = W1.T
    eye8 = jnp.eye(HEADS, dtype=_f32)
    amat = jnp.concatenate([
        (eye8[:, :, None] * att_src1[None, :, :]).reshape(HEADS, F1),
        (eye8[:, :, None] * att_dst1[None, :, :]).reshape(HEADS, F1),
    ], axis=0)
    zrow = jnp.zeros((4, NP), _f32)

    h1T, asadT = pl.pallas_call(
        _tc1_body,
        out_shape=(_sds((F1, NP)), _sds((2 * HEADS, NP))),
    )(xT, W1T, amat)
    h1p = _pack_rows(h1T, F1 // 2)

    a1 = pl.kernel(
        _sc_a1, mesh=_MESH, compiler_params=_SC_PARAMS,
        out_type=(_sds((HEADS, EP)), _sds((NW, NP))),
        scratch_types=[
            pltpu.VMEM((NP,), _f32), pltpu.VMEM((NP,), _f32),
            pltpu.VMEM((NP,), _f32),
            pltpu.VMEM((CHUNK_A,), jnp.int32), pltpu.VMEM((CHUNK_A,), jnp.int32),
            pltpu.VMEM((CHUNK_A,), _f32),
            pltpu.SemaphoreType.DMA, pltpu.SemaphoreType.DMA,
        ],
    )
    exhT, denp = a1(asadT, sd, zrow)

    def b1(half):
        return pl.kernel(
            functools.partial(_sc_b1, half=half), mesh=_MESH,
            compiler_params=_SC_PARAMS,
            out_type=_sds((128, NP)),
            scratch_types=[
                pltpu.VMEM((NP,), _f32), pltpu.VMEM((NP,), _f32),
                pltpu.VMEM((NP,), _f32), pltpu.VMEM((NP,), _f32),
                pltpu.VMEM((NP,), _f32), pltpu.VMEM((NP,), _f32),
                pltpu.VMEM((NP,), _f32), pltpu.VMEM((NP,), _f32),
                pltpu.VMEM((CHUNK_B,), jnp.int32),
                pltpu.VMEM((CHUNK_B,), _f32),
                pltpu.VMEM((CHUNK_B,), jnp.int32),
                pltpu.VMEM((CHUNK_B,), _f32),
                pltpu.SemaphoreType.DMA, pltpu.SemaphoreType.DMA,
            ],
        )(h1p, exhT, denp, sd, zrow)

    o1T = jnp.concatenate([b1(0), b1(1)], axis=0)

    W2Tp = jnp.zeros((C2P, F1), _f32).at[:C2].set(W2.T)
    a2mat = jnp.zeros((2, C2P), _f32).at[0, :C2].set(att_src2[0]).at[1, :C2].set(att_dst2[0])
    h2T, asad2 = pl.pallas_call(
        _tc2_body,
        out_shape=(_sds((C2P, NP)), _sds((2, NP))),
    )(o1T, bias1.reshape(F1, 1), W2Tp, a2mat)
    h2p = _pack_rows(h2T, C2P // 2)

    a2 = pl.kernel(
        _sc_a2, mesh=_MESH, compiler_params=_SC_PARAMS,
        out_type=(_sds((EP,)), _sds((NW, NP))),
        scratch_types=[
            pltpu.VMEM((NP,), _f32), pltpu.VMEM((NP,), _f32),
            pltpu.VMEM((NP,), _f32),
            pltpu.VMEM((EP // NW,), jnp.int32),
            pltpu.VMEM((EP // NW,), _f32),
        ],
    )
    exh2, denp2 = a2(asad2, sd, zrow)

    o2T = pl.kernel(
        _sc_b2, mesh=_MESH, compiler_params=_SC_PARAMS,
        out_type=_sds((C2P, NP)),
        scratch_types=[
            pltpu.VMEM((NP,), _f32),
            pltpu.VMEM((NP,), _f32), pltpu.VMEM((NP,), _f32),
            pltpu.VMEM((NP,), _f32), pltpu.VMEM((NP,), _f32),
            pltpu.VMEM((CHUNK_B,), jnp.int32), pltpu.VMEM((CHUNK_B,), _f32),
            pltpu.VMEM((CHUNK_B,), jnp.int32), pltpu.VMEM((CHUNK_B,), _f32),
            pltpu.SemaphoreType.DMA, pltpu.SemaphoreType.DMA,
        ],
    )(h2p, exh2, denp2, sd, zrow)

    return o2T[:C2, :N].T + bias2[None, :]
